# baseline (speedup 1.0000x reference)
import functools

import jax
import jax.numpy as jnp
from jax import lax
from jax.experimental import pallas as pl
from jax.experimental.pallas import tpu as pltpu
from jax.experimental.pallas import tpu_sc as plsc

NC = 2
NS = 16
NW = NC * NS
L = 16
SUB = 8

SC_ROWS = 6144
TC_BLOCK = 512


def _make_sc_gather(total_rows: int, row0_g: int, nrows: int, C: int, K: int):
    assert nrows % (NW * SUB) == 0 and K % L == 0 and C % 128 == 0
    assert row0_g % SUB == 0
    rows_per_w = nrows // NW
    trows_per_w = rows_per_w // SUB
    nbuf = 3 if trows_per_w % 3 == 0 else 2
    assert trows_per_w % nbuf == 0
    out_per_w = rows_per_w * K
    kchunks = K // L
    total_trows = total_rows // SUB
    trow_base = row0_g // SUB

    mesh = plsc.VectorSubcoreMesh(
        core_axis_name="c", subcore_axis_name="s",
        num_cores=NC, num_subcores=NS)

    @functools.partial(
        pl.kernel,
        mesh=mesh,
        compiler_params=pltpu.CompilerParams(needs_layout_passes=False),
        out_type=jax.ShapeDtypeStruct((nrows * K,), jnp.float32),
        scratch_types=[
            pltpu.VMEM((K,), jnp.int32),
            *[pltpu.VMEM((SUB, C), jnp.float32) for _ in range(nbuf)],
            pltpu.VMEM((out_per_w,), jnp.float32),
            *[pltpu.SemaphoreType.DMA for _ in range(nbuf)],
        ],
    )
    def k(x_hbm, idx_hbm, out_hbm, idx_v, *rest):
        blks = rest[:nbuf]
        obuf = rest[nbuf]
        sems = rest[nbuf + 1:]
        wid = lax.axis_index("s") * NC + lax.axis_index("c")
        trow0 = trow_base + wid * trows_per_w
        xv = x_hbm.reshape(total_trows, SUB, C)

        pltpu.sync_copy(idx_hbm, idx_v)

        for b in range(nbuf):
            pltpu.async_copy(xv.at[trow0 + b], blks[b], sems[b])

        def extract(blk, t):
            for s in range(SUB):
                srow = jnp.full((L,), s, dtype=jnp.int32)
                for q in range(kchunks):
                    cq = idx_v[pl.ds(q * L, L)]
                    vals = plsc.load_gather(blk, [srow, cq])
                    obuf[pl.ds((t * SUB + s) * K + q * L, L)] = vals

        def step(i, carry):
            for b in range(nbuf):
                t = i * nbuf + b
                pltpu.make_async_copy(
                    xv.at[trow0 + t], blks[b], sems[b]).wait()
                extract(blks[b], t)
                @pl.when(t + nbuf < trows_per_w)
                def _():
                    pltpu.async_copy(
                        xv.at[trow0 + t + nbuf], blks[b], sems[b])
            return carry
        lax.fori_loop(0, trows_per_w // nbuf, step, 0)

        pltpu.sync_copy(obuf, out_hbm.at[pl.ds(wid * out_per_w, out_per_w)])

    return k


def _tc_body(x_ref, sel_ref, out_ref):
    out_ref[...] = jnp.dot(
        x_ref[...], sel_ref[...], preferred_element_type=jnp.float32)


def _make_tc_gather(nrows: int, C: int, K: int):
    assert nrows % TC_BLOCK == 0
    return pl.pallas_call(
        _tc_body,
        grid=(nrows // TC_BLOCK,),
        in_specs=[
            pl.BlockSpec((TC_BLOCK, C), lambda i: (i, 0)),
            pl.BlockSpec((C, K), lambda i: (0, 0)),
        ],
        out_specs=pl.BlockSpec((TC_BLOCK, K), lambda i: (i, 0)),
        out_shape=jax.ShapeDtypeStruct((nrows, K), jnp.float32),
    )


@jax.jit
def kernel(x, indices):
    B, R, C = x.shape
    K = indices.shape[0]
    total_rows = B * R
    x2 = x.reshape(total_rows, C)
    tc_rows = total_rows - SC_ROWS

    sel = (indices[None, :] == jnp.arange(C, dtype=indices.dtype)[:, None]
           ).astype(jnp.float32)

    out_tc = _make_tc_gather(tc_rows, C, K)(x2, sel)
    sc_flat = _make_sc_gather(total_rows, tc_rows, SC_ROWS, C, K)(x, indices)
    out = jnp.concatenate([out_tc, sc_flat.reshape(SC_ROWS, K)], axis=0)
    return out.reshape(B, R, K)


# speedup vs baseline: 1.0191x; 1.0191x over previous
import functools

import jax
import jax.numpy as jnp
from jax import lax
from jax.experimental import pallas as pl
from jax.experimental.pallas import tpu as pltpu
from jax.experimental.pallas import tpu_sc as plsc

NC = 2
NS = 16
NW = NC * NS
L = 16
SUB = 8

SC_ROWS = 6656
TC_BLOCK = 512


def _make_sc_gather(total_rows: int, row0_g: int, nrows: int, C: int, K: int):
    assert nrows % (NW * SUB) == 0 and K % L == 0 and C % 128 == 0
    assert row0_g % SUB == 0
    rows_per_w = nrows // NW
    trows_per_w = rows_per_w // SUB
    nbuf = 3 if trows_per_w % 3 == 0 else 2
    assert trows_per_w % nbuf == 0
    out_per_w = rows_per_w * K
    kchunks = K // L
    total_trows = total_rows // SUB
    trow_base = row0_g // SUB

    mesh = plsc.VectorSubcoreMesh(
        core_axis_name="c", subcore_axis_name="s",
        num_cores=NC, num_subcores=NS)

    @functools.partial(
        pl.kernel,
        mesh=mesh,
        compiler_params=pltpu.CompilerParams(needs_layout_passes=False),
        out_type=jax.ShapeDtypeStruct((nrows * K,), jnp.float32),
        scratch_types=[
            pltpu.VMEM((K,), jnp.int32),
            *[pltpu.VMEM((SUB, C), jnp.float32) for _ in range(nbuf)],
            pltpu.VMEM((out_per_w,), jnp.float32),
            *[pltpu.SemaphoreType.DMA for _ in range(nbuf)],
        ],
    )
    def k(x_hbm, idx_hbm, out_hbm, idx_v, *rest):
        blks = rest[:nbuf]
        obuf = rest[nbuf]
        sems = rest[nbuf + 1:]
        wid = lax.axis_index("s") * NC + lax.axis_index("c")
        trow0 = trow_base + wid * trows_per_w
        xv = x_hbm.reshape(total_trows, SUB, C)

        pltpu.sync_copy(idx_hbm, idx_v)

        for b in range(nbuf):
            pltpu.async_copy(xv.at[trow0 + b], blks[b], sems[b])

        def extract(blk, t):
            for s in range(SUB):
                srow = jnp.full((L,), s, dtype=jnp.int32)
                for q in range(kchunks):
                    cq = idx_v[pl.ds(q * L, L)]
                    vals = plsc.load_gather(blk, [srow, cq])
                    obuf[pl.ds((t * SUB + s) * K + q * L, L)] = vals

        def step(i, carry):
            for b in range(nbuf):
                t = i * nbuf + b
                pltpu.make_async_copy(
                    xv.at[trow0 + t], blks[b], sems[b]).wait()
                extract(blks[b], t)
                @pl.when(t + nbuf < trows_per_w)
                def _():
                    pltpu.async_copy(
                        xv.at[trow0 + t + nbuf], blks[b], sems[b])
            return carry
        lax.fori_loop(0, trows_per_w // nbuf, step, 0)

        pltpu.sync_copy(obuf, out_hbm.at[pl.ds(wid * out_per_w, out_per_w)])

    return k


def _tc_body(x_ref, sel_ref, out_ref):
    out_ref[...] = jnp.dot(
        x_ref[...], sel_ref[...], preferred_element_type=jnp.float32)


def _make_tc_gather(nrows: int, C: int, K: int):
    assert nrows % TC_BLOCK == 0
    return pl.pallas_call(
        _tc_body,
        grid=(nrows // TC_BLOCK,),
        in_specs=[
            pl.BlockSpec((TC_BLOCK, C), lambda i: (i, 0)),
            pl.BlockSpec((C, K), lambda i: (0, 0)),
        ],
        out_specs=pl.BlockSpec((TC_BLOCK, K), lambda i: (i, 0)),
        out_shape=jax.ShapeDtypeStruct((nrows, K), jnp.float32),
    )


@jax.jit
def kernel(x, indices):
    B, R, C = x.shape
    K = indices.shape[0]
    total_rows = B * R
    x2 = x.reshape(total_rows, C)
    tc_rows = total_rows - SC_ROWS

    sel = (indices[None, :] == jnp.arange(C, dtype=indices.dtype)[:, None]
           ).astype(jnp.float32)

    out_tc = _make_tc_gather(tc_rows, C, K)(x2, sel)
    sc_flat = _make_sc_gather(total_rows, tc_rows, SC_ROWS, C, K)(x, indices)
    out = jnp.concatenate([out_tc, sc_flat.reshape(SC_ROWS, K)], axis=0)
    return out.reshape(B, R, K)
